# baseline (speedup 1.0000x reference)
import functools

import jax
import jax.numpy as jnp
from jax import lax
from jax.experimental import pallas as pl
from jax.experimental.pallas import tpu as pltpu
from jax.experimental.pallas import tpu_sc as plsc

N_NODES = 50000
DEG = 16
D_FEAT = 256
HIDDEN = 128
N_CAT = 40
B = 1024

NC = 2
NS = 16
NW = NC * NS
IDS_PER_TILE = B // NW
P = IDS_PER_TILE * DEG
CHUNK = 128
NCHUNK = P // CHUNK


def _mm_body(x_ref, w_ref, o_ref):
    o_ref[...] = lax.dot_general(
        x_ref[...], w_ref[...],
        dimension_numbers=(((1,), (1,)), ((), ())),
        preferred_element_type=jnp.float32)


def _precompute_g(feats, W0):
    n_blocks = 25
    rows = N_NODES // n_blocks
    return pl.pallas_call(
        _mm_body,
        grid=(n_blocks,),
        in_specs=[
            pl.BlockSpec((rows, D_FEAT), lambda i: (i, 0)),
            pl.BlockSpec((HIDDEN, D_FEAT), lambda i: (0, 0)),
        ],
        out_specs=pl.BlockSpec((rows, HIDDEN), lambda i: (i, 0)),
        out_shape=jax.ShapeDtypeStruct((N_NODES, HIDDEN), jnp.float32),
    )(feats, W0)


N_PAD = 50048


def _sc_chase_body(ids_hbm, edge_flat_hbm, out_hbm,
                   eidx, nb1f, idxbuf, nb2t, sem, sem2):
    wid = lax.axis_index("s") * NC + lax.axis_index("c")
    base = wid * IDS_PER_TILE
    iota = lax.iota(jnp.int32, 16)

    def _expand(q, carry):
        eidx[pl.ds(q * DEG, DEG)] = jnp.full((16,), base + q, jnp.int32)
        return carry
    lax.fori_loop(0, IDS_PER_TILE, _expand, 0)
    exp = [
        pltpu.async_copy(ids_hbm.at[eidx.at[pl.ds(c * CHUNK, CHUNK)]],
                         nb1f.at[pl.ds(c * CHUNK, CHUNK)], sem)
        for c in range(NCHUNK)
    ]
    for d in exp:
        d.wait()
    def _idx1(q, carry):
        v = nb1f[pl.ds(q * DEG, DEG)]
        eidx[pl.ds(q * DEG, DEG)] = v + iota * N_PAD
        return carry
    lax.fori_loop(0, IDS_PER_TILE, _idx1, 0)

    h1 = [
        pltpu.async_copy(edge_flat_hbm.at[eidx.at[pl.ds(c * CHUNK, CHUNK)]],
                         nb1f.at[pl.ds(c * CHUNK, CHUNK)], sem)
        for c in range(NCHUNK)
    ]
    for d in h1:
        d.wait()

    def _build(q, carry):
        v = nb1f[pl.ds(q * DEG, DEG)]
        for k in range(DEG):
            idxbuf[pl.ds(k * P + q * DEG, DEG)] = v + k * N_PAD
        return carry
    lax.fori_loop(0, IDS_PER_TILE, _build, 0)

    h2 = [
        pltpu.async_copy(
            edge_flat_hbm.at[idxbuf.at[pl.ds(k * P + c * CHUNK, CHUNK)]],
            nb2t.at[pl.ds(k * P + c * CHUNK, CHUNK)], sem2)
        for k in range(DEG)
        for c in range(NCHUNK)
    ]
    for d in h2:
        d.wait()
    pltpu.sync_copy(nb2t, out_hbm.at[pl.ds(wid * DEG * P, DEG * P)])


def _sc_gather_body(nb2t_hbm, g_hbm, b0_hbm, out_hbm,
                    nb2t, acc, pooled, b0v, sem, *csems):
    wid = lax.axis_index("s") * NC + lax.axis_index("c")
    base = wid * IDS_PER_TILE
    pltpu.sync_copy(nb2t_hbm.at[pl.ds(wid * DEG * P, DEG * P)], nb2t)
    pltpu.sync_copy(b0_hbm, b0v)

    init = [
        pltpu.async_copy(g_hbm.at[nb2t.at[pl.ds(c * CHUNK, CHUNK)]],
                         acc.at[pl.ds(c * CHUNK, CHUNK)], sem)
        for c in range(NCHUNK)
    ]
    for d in init:
        d.wait()
    adds = [
        pltpu.async_copy(g_hbm.at[nb2t.at[pl.ds(k * P + c * CHUNK, CHUNK)]],
                         acc.at[pl.ds(c * CHUNK, CHUNK)], csems[c], add=True)
        for k in range(1, DEG)
        for c in range(NCHUNK)
    ]
    scale2 = jnp.float32(1.0 / (DEG * DEG))
    for c in range(NCHUNK):
        for k in range(1, DEG):
            adds[(k - 1) * NCHUNK + c].wait()
        bpc = CHUNK // DEG
        def _pool(b, carry, c=c):
            p0 = c * CHUNK + b * DEG
            for v in range(HIDDEN // 16):
                col = pl.ds(v * 16, 16)
                b0g = b0v[col] * jnp.float32(DEG)
                tot = jnp.maximum(acc[p0, col] + b0g, 0.0)
                for j in range(1, DEG):
                    tot = tot + jnp.maximum(acc[p0 + j, col] + b0g, 0.0)
                pooled[c * bpc + b, col] = tot * scale2
            return carry
        lax.fori_loop(0, bpc, _pool, 0)

    pltpu.sync_copy(pooled, out_hbm.at[pl.ds(base, IDS_PER_TILE)])


def _sc_chase(ids, edge_flat):
    mesh = plsc.VectorSubcoreMesh(core_axis_name="c", subcore_axis_name="s")
    f = functools.partial(
        pl.kernel,
        out_type=jax.ShapeDtypeStruct((NW * DEG * P,), jnp.int32),
        mesh=mesh,
        scratch_types=[
            pltpu.VMEM((P,), jnp.int32),
            pltpu.VMEM((P,), jnp.int32),
            pltpu.VMEM((DEG * P,), jnp.int32),
            pltpu.VMEM((DEG * P,), jnp.int32),
            pltpu.SemaphoreType.DMA,
            pltpu.SemaphoreType.DMA,
        ],
    )(_sc_chase_body)
    return f(ids, edge_flat)


def _sc_gather_sum(nb2t_all, g, b0):
    mesh = plsc.VectorSubcoreMesh(core_axis_name="c", subcore_axis_name="s")
    f = functools.partial(
        pl.kernel,
        out_type=jax.ShapeDtypeStruct((B, HIDDEN), jnp.float32),
        mesh=mesh,
        scratch_types=[
            pltpu.VMEM((DEG * P,), jnp.int32),
            pltpu.VMEM((P, HIDDEN), jnp.float32),
            pltpu.VMEM((IDS_PER_TILE, HIDDEN), jnp.float32),
            pltpu.VMEM((HIDDEN,), jnp.float32),
            pltpu.SemaphoreType.DMA,
        ] + [pltpu.SemaphoreType.DMA] * NCHUNK,
    )(_sc_gather_body)
    return f(nb2t_all, g, b0)


def _tail_body(p_ref, w1_ref, b1_ref, o_ref):
    pooled = p_ref[...]
    logits = lax.dot_general(
        w1_ref[...], pooled,
        dimension_numbers=(((1,), (1,)), ((), ())),
        preferred_element_type=jnp.float32) + b1_ref[...]
    m = jnp.max(logits, axis=0, keepdims=True)
    e = jnp.exp(logits - m)
    lse = jnp.log(jnp.sum(e, axis=0, keepdims=True)) + m
    o_ref[...] = logits - lse


def _tail(pooled, W1, b1):
    out_t = pl.pallas_call(
        _tail_body,
        in_specs=[
            pl.BlockSpec((B, HIDDEN), lambda: (0, 0)),
            pl.BlockSpec((N_CAT, HIDDEN), lambda: (0, 0)),
            pl.BlockSpec((N_CAT, 1), lambda: (0, 0)),
        ],
        out_specs=pl.BlockSpec((N_CAT, B), lambda: (0, 0)),
        out_shape=jax.ShapeDtypeStruct((N_CAT, B), jnp.float32),
    )(pooled, W1, b1.reshape(N_CAT, 1))
    return out_t.T


def kernel(ids, feats, edge_dict, W0, b0, W1, b1):
    edge_flat = jnp.pad(edge_dict.T, ((0, 0), (0, N_PAD - N_NODES))).reshape(-1)
    nb2t_all = _sc_chase(ids, edge_flat)
    g = _precompute_g(feats, W0)
    pooled = _sc_gather_sum(nb2t_all, g, b0)
    return _tail(pooled, W1, b1)


# speedup vs baseline: 1.0341x; 1.0341x over previous
import functools

import jax
import jax.numpy as jnp
from jax import lax
from jax.experimental import pallas as pl
from jax.experimental.pallas import tpu as pltpu
from jax.experimental.pallas import tpu_sc as plsc

N_NODES = 50000
DEG = 16
D_FEAT = 256
HIDDEN = 128
N_CAT = 40
B = 1024

NC = 2
NS = 16
NW = NC * NS
IDS_PER_TILE = B // NW
P = IDS_PER_TILE * DEG
CHUNK = 128
NCHUNK = P // CHUNK


def _mm_body(x_ref, w_ref, o_ref):
    o_ref[...] = lax.dot_general(
        x_ref[...], w_ref[...],
        dimension_numbers=(((1,), (1,)), ((), ())),
        preferred_element_type=jnp.float32)


def _precompute_g(feats, W0):
    n_blocks = 25
    rows = N_NODES // n_blocks
    return pl.pallas_call(
        _mm_body,
        grid=(n_blocks,),
        in_specs=[
            pl.BlockSpec((rows, D_FEAT), lambda i: (i, 0)),
            pl.BlockSpec((HIDDEN, D_FEAT), lambda i: (0, 0)),
        ],
        out_specs=pl.BlockSpec((rows, HIDDEN), lambda i: (i, 0)),
        out_shape=jax.ShapeDtypeStruct((N_NODES, HIDDEN), jnp.float32),
    )(feats, W0)


N_PAD = 50048


def _sc_chase_body(ids_hbm, edge_flat_hbm, out_hbm,
                   eidx, nb1f, idxbuf, nb2t, sem, sem2):
    wid = lax.axis_index("s") * NC + lax.axis_index("c")
    base = wid * IDS_PER_TILE
    iota = lax.iota(jnp.int32, 16)

    def _expand(q, carry):
        eidx[pl.ds(q * DEG, DEG)] = jnp.full((16,), base + q, jnp.int32)
        return carry
    lax.fori_loop(0, IDS_PER_TILE, _expand, 0)
    exp = [
        pltpu.async_copy(ids_hbm.at[eidx.at[pl.ds(c * CHUNK, CHUNK)]],
                         nb1f.at[pl.ds(c * CHUNK, CHUNK)], sem)
        for c in range(NCHUNK)
    ]
    for d in exp:
        d.wait()
    def _idx1(q, carry):
        v = nb1f[pl.ds(q * DEG, DEG)]
        eidx[pl.ds(q * DEG, DEG)] = v + iota * N_PAD
        return carry
    lax.fori_loop(0, IDS_PER_TILE, _idx1, 0)

    h1 = [
        pltpu.async_copy(edge_flat_hbm.at[eidx.at[pl.ds(c * CHUNK, CHUNK)]],
                         nb1f.at[pl.ds(c * CHUNK, CHUNK)], sem)
        for c in range(NCHUNK)
    ]
    for d in h1:
        d.wait()

    def _build(q, carry):
        v = nb1f[pl.ds(q * DEG, DEG)]
        for k in range(DEG):
            idxbuf[pl.ds(k * P + q * DEG, DEG)] = v + k * N_PAD
        return carry
    lax.fori_loop(0, IDS_PER_TILE, _build, 0)

    h2 = [
        pltpu.async_copy(
            edge_flat_hbm.at[idxbuf.at[pl.ds(k * P + c * CHUNK, CHUNK)]],
            nb2t.at[pl.ds(k * P + c * CHUNK, CHUNK)], sem2)
        for k in range(DEG)
        for c in range(NCHUNK)
    ]
    for d in h2:
        d.wait()
    pltpu.sync_copy(nb2t, out_hbm.at[pl.ds(wid * DEG * P, DEG * P)])


def _sc_gather_body(nb2t_hbm, g_hbm, b0_hbm, out_hbm,
                    nb2t, acc, pooled, b0v, sem, *csems):
    wid = lax.axis_index("s") * NC + lax.axis_index("c")
    base = wid * IDS_PER_TILE
    pltpu.sync_copy(nb2t_hbm.at[pl.ds(wid * DEG * P, DEG * P)], nb2t)
    pltpu.sync_copy(b0_hbm, b0v)

    init = [
        pltpu.async_copy(g_hbm.at[nb2t.at[pl.ds(c * CHUNK, CHUNK)]],
                         acc.at[pl.ds(c * CHUNK, CHUNK)], sem)
        for c in range(NCHUNK)
    ]
    for d in init:
        d.wait()
    adds = [
        pltpu.async_copy(g_hbm.at[nb2t.at[pl.ds(k * P + c * CHUNK, CHUNK)]],
                         acc.at[pl.ds(c * CHUNK, CHUNK)], csems[c], add=True)
        for c in range(NCHUNK)
        for k in range(1, DEG)
    ]
    scale2 = jnp.float32(1.0 / (DEG * DEG))
    for c in range(NCHUNK):
        for k in range(1, DEG):
            adds[c * (DEG - 1) + (k - 1)].wait()
        bpc = CHUNK // DEG
        def _pool(b, carry, c=c):
            p0 = c * CHUNK + b * DEG
            for v in range(HIDDEN // 16):
                col = pl.ds(v * 16, 16)
                b0g = b0v[col] * jnp.float32(DEG)
                tot = jnp.maximum(acc[p0, col] + b0g, 0.0)
                for j in range(1, DEG):
                    tot = tot + jnp.maximum(acc[p0 + j, col] + b0g, 0.0)
                pooled[c * bpc + b, col] = tot * scale2
            return carry
        lax.fori_loop(0, bpc, _pool, 0)

    pltpu.sync_copy(pooled, out_hbm.at[pl.ds(base, IDS_PER_TILE)])


def _sc_chase(ids, edge_flat):
    mesh = plsc.VectorSubcoreMesh(core_axis_name="c", subcore_axis_name="s")
    f = functools.partial(
        pl.kernel,
        out_type=jax.ShapeDtypeStruct((NW * DEG * P,), jnp.int32),
        mesh=mesh,
        scratch_types=[
            pltpu.VMEM((P,), jnp.int32),
            pltpu.VMEM((P,), jnp.int32),
            pltpu.VMEM((DEG * P,), jnp.int32),
            pltpu.VMEM((DEG * P,), jnp.int32),
            pltpu.SemaphoreType.DMA,
            pltpu.SemaphoreType.DMA,
        ],
    )(_sc_chase_body)
    return f(ids, edge_flat)


def _sc_gather_sum(nb2t_all, g, b0):
    mesh = plsc.VectorSubcoreMesh(core_axis_name="c", subcore_axis_name="s")
    f = functools.partial(
        pl.kernel,
        out_type=jax.ShapeDtypeStruct((B, HIDDEN), jnp.float32),
        mesh=mesh,
        scratch_types=[
            pltpu.VMEM((DEG * P,), jnp.int32),
            pltpu.VMEM((P, HIDDEN), jnp.float32),
            pltpu.VMEM((IDS_PER_TILE, HIDDEN), jnp.float32),
            pltpu.VMEM((HIDDEN,), jnp.float32),
            pltpu.SemaphoreType.DMA,
        ] + [pltpu.SemaphoreType.DMA] * NCHUNK,
    )(_sc_gather_body)
    return f(nb2t_all, g, b0)


def _tail_body(p_ref, w1_ref, b1_ref, o_ref):
    pooled = p_ref[...]
    logits = lax.dot_general(
        w1_ref[...], pooled,
        dimension_numbers=(((1,), (1,)), ((), ())),
        preferred_element_type=jnp.float32) + b1_ref[...]
    m = jnp.max(logits, axis=0, keepdims=True)
    e = jnp.exp(logits - m)
    lse = jnp.log(jnp.sum(e, axis=0, keepdims=True)) + m
    o_ref[...] = logits - lse


def _tail(pooled, W1, b1):
    out_t = pl.pallas_call(
        _tail_body,
        in_specs=[
            pl.BlockSpec((B, HIDDEN), lambda: (0, 0)),
            pl.BlockSpec((N_CAT, HIDDEN), lambda: (0, 0)),
            pl.BlockSpec((N_CAT, 1), lambda: (0, 0)),
        ],
        out_specs=pl.BlockSpec((N_CAT, B), lambda: (0, 0)),
        out_shape=jax.ShapeDtypeStruct((N_CAT, B), jnp.float32),
    )(pooled, W1, b1.reshape(N_CAT, 1))
    return out_t.T


def kernel(ids, feats, edge_dict, W0, b0, W1, b1):
    edge_flat = jnp.pad(edge_dict.T, ((0, 0), (0, N_PAD - N_NODES))).reshape(-1)
    nb2t_all = _sc_chase(ids, edge_flat)
    g = _precompute_g(feats, W0)
    pooled = _sc_gather_sum(nb2t_all, g, b0)
    return _tail(pooled, W1, b1)
